# baseline (speedup 1.0000x reference)
import functools

import jax
import jax.numpy as jnp
from jax import lax
from jax.experimental import pallas as pl
from jax.experimental.pallas import tpu as pltpu
from jax.experimental.pallas import tpu_sc as plsc

N_FIELDS = 26
VOCAB = 100000
EMB = 6
EMBP = 8
N_NUMERIC = 13
BATCH = 4096
D_IN = N_FIELDS * EMB + N_NUMERIC
L1 = 1024
L2 = 206

NC, NS = 2, 16
NW = NC * NS
NLOOK = BATCH * N_FIELDS
L_PER_W = NLOOK // NW

RCHUNK = 14336
NBLK = 7
FPITCH = NBLK * RCHUNK
NROWS = N_FIELDS * FPITCH


def _tc_pad_body(t_ref, o_ref):
    x = t_ref[0]
    o_ref[...] = jnp.concatenate(
        [x, jnp.zeros((RCHUNK, EMBP - EMB), jnp.float32)], axis=1)


def _tc_pad(tables):
    grid = (N_FIELDS, NBLK)
    return pl.pallas_call(
        _tc_pad_body,
        grid=grid,
        in_specs=[pl.BlockSpec((1, RCHUNK, EMB), lambda f, c: (f, c, 0))],
        out_specs=pl.BlockSpec((RCHUNK, EMBP), lambda f, c: (f * NBLK + c, 0)),
        out_shape=jax.ShapeDtypeStruct((NROWS, EMBP), jnp.float32),
    )(tables)


def _sc_gather(t8, ridx):
    mesh = plsc.VectorSubcoreMesh(core_axis_name="c", subcore_axis_name="s")

    @functools.partial(
        pl.kernel,
        mesh=mesh,
        compiler_params=pltpu.CompilerParams(use_tc_tiling_on_sc=False),
        out_type=jax.ShapeDtypeStruct((NLOOK, EMBP), jnp.float32),
        scratch_types=[
            pltpu.VMEM((L_PER_W,), jnp.int32),
            pltpu.VMEM((L_PER_W, EMBP), jnp.float32),
            pltpu.SemaphoreType.DMA,
        ],
    )
    def k(t_hbm, i_hbm, o_hbm, idx_v, rows_v, sem):
        wid = lax.axis_index("s") * NC + lax.axis_index("c")
        base = wid * L_PER_W
        pltpu.sync_copy(i_hbm.at[pl.ds(base, L_PER_W)], idx_v)
        pltpu.async_copy(t_hbm.at[idx_v], rows_v, sem).wait()
        pltpu.sync_copy(rows_v, o_hbm.at[pl.ds(base, L_PER_W)])

    return k(t8, ridx)


def _mlp_body(f_ref, w1_ref, b1_ref, w2_ref, b2_ref, w3_ref, b3_ref,
              h2_ref, out_ref):
    f = f_ref[...]
    h1 = jnp.maximum(
        jnp.dot(f, w1_ref[...], preferred_element_type=jnp.float32)
        + b1_ref[...], 0.0)
    h2 = jnp.maximum(
        jnp.dot(h1, w2_ref[...], preferred_element_type=jnp.float32)
        + b2_ref[...], 0.0)
    h2_ref[...] = h2
    z = jnp.dot(h2, w3_ref[...], preferred_element_type=jnp.float32) + b3_ref[...]
    out_ref[...] = jax.nn.sigmoid(z)


def _mlp(feats, W1, b1, W2, b2, W3, b3):
    BB = 512
    grid = (BATCH // BB,)
    h2, out = pl.pallas_call(
        _mlp_body,
        grid=grid,
        in_specs=[
            pl.BlockSpec((BB, D_IN), lambda i: (i, 0)),
            pl.BlockSpec((D_IN, L1), lambda i: (0, 0)),
            pl.BlockSpec((1, L1), lambda i: (0, 0)),
            pl.BlockSpec((L1, L2), lambda i: (0, 0)),
            pl.BlockSpec((1, L2), lambda i: (0, 0)),
            pl.BlockSpec((L2, 1), lambda i: (0, 0)),
            pl.BlockSpec((1, 1), lambda i: (0, 0)),
        ],
        out_specs=[
            pl.BlockSpec((BB, L2), lambda i: (i, 0)),
            pl.BlockSpec((BB, 1), lambda i: (i, 0)),
        ],
        out_shape=[
            jax.ShapeDtypeStruct((BATCH, L2), jnp.float32),
            jax.ShapeDtypeStruct((BATCH, 1), jnp.float32),
        ],
    )(feats, W1, b1.reshape(1, L1), W2, b2.reshape(1, L2), W3,
      b3.reshape(1, 1))
    return h2, out


def kernel(X, tables, W1, b1, W2, b2, W3, b3):
    idx = X[:, :N_FIELDS].astype(jnp.int32)
    ridx = (idx + jnp.arange(N_FIELDS, dtype=jnp.int32) * FPITCH).reshape(-1)
    t8 = _tc_pad(tables)
    rows = _sc_gather(t8, ridx)
    embeds_flat = rows[:, :EMB].reshape(BATCH, N_FIELDS * EMB)
    feats = jnp.concatenate([embeds_flat, X[:, N_FIELDS:]], axis=1)
    h2, out = _mlp(feats, W1, b1, W2, b2, W3, b3)
    return (embeds_flat, h2, out)


# speedup vs baseline: 1.6786x; 1.6786x over previous
import functools

import jax
import jax.numpy as jnp
from jax import lax
from jax.experimental import pallas as pl
from jax.experimental.pallas import tpu as pltpu
from jax.experimental.pallas import tpu_sc as plsc

N_FIELDS = 26
VOCAB = 100000
EMB = 6
N_NUMERIC = 13
BATCH = 4096
D_IN = N_FIELDS * EMB + N_NUMERIC
L1 = 1024
L2 = 206

NC, NS = 2, 16
NW = NC * NS
BPW = BATCH // NW


NELEM = BATCH * N_FIELDS * EMB
E_PER_W = NELEM // NW


def _sc_gather(tables, gidx6):
    mesh = plsc.VectorSubcoreMesh(core_axis_name="c", subcore_axis_name="s")

    @functools.partial(
        pl.kernel,
        mesh=mesh,
        compiler_params=pltpu.CompilerParams(use_tc_tiling_on_sc=False),
        out_type=jax.ShapeDtypeStruct((NELEM,), jnp.float32),
        scratch_types=[
            pltpu.VMEM((E_PER_W,), jnp.int32),
            pltpu.VMEM((E_PER_W,), jnp.float32),
            pltpu.SemaphoreType.DMA,
        ],
    )
    def k(t_hbm, i_hbm, o_hbm, idx_v, vals_v, sem):
        wid = lax.axis_index("s") * NC + lax.axis_index("c")
        base = wid * E_PER_W
        pltpu.sync_copy(i_hbm.at[pl.ds(base, E_PER_W)], idx_v)
        pltpu.async_copy(t_hbm.at[idx_v], vals_v, sem).wait()
        pltpu.sync_copy(vals_v, o_hbm.at[pl.ds(base, E_PER_W)])

    return k(tables, gidx6)


def _mlp_body(f_ref, w1_ref, b1_ref, w2_ref, b2_ref, w3_ref, b3_ref,
              h2_ref, out_ref):
    f = f_ref[...]
    h1 = jnp.maximum(
        jnp.dot(f, w1_ref[...], preferred_element_type=jnp.float32)
        + b1_ref[...], 0.0)
    h2 = jnp.maximum(
        jnp.dot(h1, w2_ref[...], preferred_element_type=jnp.float32)
        + b2_ref[...], 0.0)
    h2_ref[...] = h2
    z = jnp.dot(h2, w3_ref[...], preferred_element_type=jnp.float32) + b3_ref[...]
    out_ref[...] = jax.nn.sigmoid(z)


def _mlp(feats, W1, b1, W2, b2, W3, b3):
    BB = 512
    grid = (BATCH // BB,)
    h2, out = pl.pallas_call(
        _mlp_body,
        grid=grid,
        in_specs=[
            pl.BlockSpec((BB, D_IN), lambda i: (i, 0)),
            pl.BlockSpec((D_IN, L1), lambda i: (0, 0)),
            pl.BlockSpec((1, L1), lambda i: (0, 0)),
            pl.BlockSpec((L1, L2), lambda i: (0, 0)),
            pl.BlockSpec((1, L2), lambda i: (0, 0)),
            pl.BlockSpec((L2, 1), lambda i: (0, 0)),
            pl.BlockSpec((1, 1), lambda i: (0, 0)),
        ],
        out_specs=[
            pl.BlockSpec((BB, L2), lambda i: (i, 0)),
            pl.BlockSpec((BB, 1), lambda i: (i, 0)),
        ],
        out_shape=[
            jax.ShapeDtypeStruct((BATCH, L2), jnp.float32),
            jax.ShapeDtypeStruct((BATCH, 1), jnp.float32),
        ],
    )(feats, W1, b1.reshape(1, L1), W2, b2.reshape(1, L2), W3,
      b3.reshape(1, 1))
    return h2, out


def kernel(X, tables, W1, b1, W2, b2, W3, b3):
    idx = X[:, :N_FIELDS].astype(jnp.int32)
    gidx = idx + jnp.arange(N_FIELDS, dtype=jnp.int32) * VOCAB
    gidx6 = (gidx.reshape(-1)[:, None] * EMB
             + jnp.arange(EMB, dtype=jnp.int32)).reshape(-1)
    vals = _sc_gather(tables.reshape(-1), gidx6)
    embeds_flat = vals.reshape(BATCH, N_FIELDS * EMB)
    feats = jnp.concatenate([embeds_flat, X[:, N_FIELDS:]], axis=1)
    h2, out = _mlp(feats, W1, b1, W2, b2, W3, b3)
    return (embeds_flat, h2, out)
